# baseline (speedup 1.0000x reference)
import jax
import jax.numpy as jnp
from jax import lax
from jax.experimental import pallas as pl
from jax.experimental.pallas import tpu as pltpu
from jax.experimental.pallas import tpu_sc as plsc

N_CHANNELS = 384
N_NEIGHB = 64
N_SPIKES = 1_000_000
N_QUERY_CH = 96
MIN_COVERAGE = 0.9

L = 16
NC = 2
NS = 16
NW = NC * NS
NV = N_SPIKES // L
BASE_V = NV // NW
REM_V = NV % NW
CHUNK_V = 512
CW = CHUNK_V * L
N_CHUNKS = -(-(BASE_V + 1) // CHUNK_V)
TAIL_V = (BASE_V + 1) - (N_CHUNKS - 1) * CHUNK_V
CHUNK_SIZES = [CHUNK_V] * (N_CHUNKS - 1) + [TAIL_V]
U = 8
NPT = N_NEIGHB // NS
QC = N_CHANNELS // L


def _sc_body(indT_hbm, ids_hbm, ch_hbm, pc_hbm,
             cov_hbm, cvd_hbm, nsp_hbm, out_hbm,
             ind4_v, ch_v, pc_v, m_v, slot_v, sm_v,
             small_v, table_v, idbufs, obufs, shared_sm,
             sins, souts, sa):
    cid = lax.axis_index("c")
    sid = lax.axis_index("s")
    iota = lax.iota(jnp.int32, L)
    zero = jnp.zeros((L,), jnp.float32)

    w = sid * NC + cid
    n_w = BASE_V + jnp.where(w < REM_V, 1, 0)
    s_w = BASE_V * w + jnp.minimum(w, REM_V)

    def chunk_base(i):
        if i < N_CHUNKS - 1:
            coff = jnp.int32(i * CHUNK_V)
        else:
            coff = n_w - TAIL_V
        return (s_w + coff) * L

    ind_d = pltpu.async_copy(indT_hbm.at[pl.ds(sid * NPT, NPT)], ind4_v, sa)
    ch_d = pltpu.async_copy(ch_hbm, ch_v, sa)
    in_d = [pltpu.async_copy(
        ids_hbm.at[pl.ds(chunk_base(i), CHUNK_SIZES[i] * L)],
        idbufs[i].at[pl.ds(0, CHUNK_SIZES[i] * L)], sins[i])
            for i in range(N_CHUNKS)]

    @pl.when((sid == 0) & (cid == 0))
    def _load_pc():
        pltpu.sync_copy(pc_hbm, pc_v)

    for q in range(QC):
        m_v[pl.ds(q * L, L)] = zero
    ones = jnp.ones((L,), jnp.float32)
    ind_d.wait()
    ch_d.wait()
    for g in range(N_QUERY_CH // L):
        plsc.addupdate_scatter(m_v, [ch_v[pl.ds(g * L, L)]], ones)

    slot = zero
    for n in range(NPT):
        @plsc.parallel_loop(0, QC, step=1, unroll=6, carry=(zero, zero))
        def accs(q, c):
            acc_c, acc_s = c
            row = ind4_v[n, pl.ds(q * L, L)]
            return acc_c + row, acc_s + m_v[pl.ds(q * L, L)] * row
        cnt_n = jnp.sum(accs[0])
        ssum_n = jnp.sum(accs[1])
        cov_vec = (jnp.full((L,), ssum_n, jnp.float32)
                   / jnp.full((L,), cnt_n, jnp.float32))
        masked_vec = jnp.where(cov_vec >= MIN_COVERAGE, cov_vec,
                               jnp.float32(0.0))
        slot = jnp.where(iota == n, cov_vec, slot)
        slot = jnp.where(iota == NPT + n, masked_vec, slot)
    slot_v[pl.ds(0, L)] = slot
    pltpu.sync_copy(slot_v, shared_sm.at[pl.ds(sid * L, L)])
    plsc.subcore_barrier()
    pltpu.sync_copy(shared_sm, sm_v)

    for jj in range(N_NEIGHB // L):
        j = jj * L + iota
        table_v[pl.ds(jj * L, L)] = plsc.load_gather(
            sm_v, [(j >> 2) * L + NPT + (j & (NPT - 1))])

    @pl.when((sid == 0) & (cid == 0))
    def _small_compute():
        nsp = jnp.int32(0)
        for jj in range(N_NEIGHB // L):
            j = jj * L + iota
            covj = plsc.load_gather(sm_v, [(j >> 2) * L + (j & (NPT - 1))])
            cvdj = covj >= MIN_COVERAGE
            pc = pc_v[pl.ds(jj * L, L)]
            nsp = nsp + jnp.sum(jnp.where(cvdj, pc, jnp.int32(0)))
            small_v[pl.ds(jj * L, L)] = covj
            small_v[pl.ds(N_NEIGHB + jj * L, L)] = jnp.where(
                cvdj, jnp.float32(1.0), jnp.float32(0.0))
        small_v[pl.ds(2 * N_NEIGHB, L)] = jnp.full(
            (L,), nsp, jnp.int32).astype(jnp.float32)

    out_d = [None] * N_CHUNKS
    for i in range(N_CHUNKS):
        ib = idbufs[i]
        ob = obufs[i]
        in_d[i].wait()

        @plsc.parallel_loop(0, CHUNK_SIZES[i], step=1, unroll=U)
        def _g(k, ib=ib, ob=ob):
            off = k * L
            ob[pl.ds(off, L)] = plsc.load_gather(
                table_v, [ib[pl.ds(off, L)]])

        out_d[i] = pltpu.async_copy(
            ob.at[pl.ds(0, CHUNK_SIZES[i] * L)],
            out_hbm.at[pl.ds(chunk_base(i), CHUNK_SIZES[i] * L)], souts[i])

    @pl.when((sid == 0) & (cid == 0))
    def _write_small():
        pltpu.sync_copy(small_v.at[pl.ds(0, N_NEIGHB)], cov_hbm)
        pltpu.sync_copy(small_v.at[pl.ds(N_NEIGHB, N_NEIGHB)], cvd_hbm)
        pltpu.sync_copy(small_v.at[pl.ds(2 * N_NEIGHB, L)], nsp_hbm)

    for i in range(N_CHUNKS):
        out_d[i].wait()


@jax.jit
def _run(indT, ids, ch, pc):
    mesh = plsc.VectorSubcoreMesh(core_axis_name="c", subcore_axis_name="s",
                                  num_cores=NC, num_subcores=NS)
    f = pl.kernel(
        _sc_body,
        out_type=(
            jax.ShapeDtypeStruct((N_NEIGHB,), jnp.float32),
            jax.ShapeDtypeStruct((N_NEIGHB,), jnp.float32),
            jax.ShapeDtypeStruct((L,), jnp.float32),
            jax.ShapeDtypeStruct((N_SPIKES,), jnp.float32),
        ),
        mesh=mesh,
        compiler_params=pltpu.CompilerParams(needs_layout_passes=False),
        scratch_types=(
            pltpu.VMEM((NPT, N_CHANNELS), jnp.float32),
            pltpu.VMEM((N_QUERY_CH,), jnp.int32),
            pltpu.VMEM((N_NEIGHB,), jnp.int32),
            pltpu.VMEM((N_CHANNELS,), jnp.float32),
            pltpu.VMEM((L,), jnp.float32),
            pltpu.VMEM((NS * L,), jnp.float32),
            pltpu.VMEM((2 * N_NEIGHB + L,), jnp.float32),
            pltpu.VMEM((N_NEIGHB,), jnp.float32),
            [pltpu.VMEM((CW,), jnp.int32)] * N_CHUNKS,
            [pltpu.VMEM((CW,), jnp.float32)] * N_CHUNKS,
            pltpu.VMEM_SHARED((NS * L,), jnp.float32),
            [pltpu.SemaphoreType.DMA] * N_CHUNKS,
            [pltpu.SemaphoreType.DMA] * N_CHUNKS,
            pltpu.SemaphoreType.DMA,
        ),
    )
    return f(indT, ids, ch, pc)


def kernel(indicators, neighborhood_ids, channels, popcounts):
    cov, cvd, nsp, spike_cov = _run(
        indicators.astype(jnp.float32).T, neighborhood_ids.astype(jnp.int32),
        channels.astype(jnp.int32), popcounts.astype(jnp.int32))
    covered = cvd != 0.0
    n_spikes_covered = nsp[0].astype(jnp.int32)
    return cov, covered, n_spikes_covered, spike_cov
